# baseline (speedup 1.0000x reference)
import functools

import jax
import jax.numpy as jnp
from jax.experimental import pallas as pl
from jax.experimental.pallas import tpu as pltpu

EPS2 = 1e-4
NEPS = 1e-12
TAU = 0.1
GAMMA = 64.0
KNN = 16

_HI = jax.lax.Precision.HIGHEST


def _dot(a, b, prec=None):
    return jax.lax.dot_general(a, b, (((1,), (0,)), ((), ())),
                               precision=prec,
                               preferred_element_type=jnp.float32)


def _dot_t(a, b, prec=None):
    return jax.lax.dot_general(a, b, (((0,), (0,)), ((), ())),
                               precision=prec,
                               preferred_element_type=jnp.float32)


def _dot_nt(a, b, prec=None):
    return jax.lax.dot_general(a, b, (((1,), (1,)), ((), ())),
                               precision=prec,
                               preferred_element_type=jnp.float32)


def _kth_bits(bits, k, axis):
    shp = list(bits.shape)
    shp[axis] = 1
    res = jnp.zeros(shp, jnp.int32)
    kf = jnp.float32(k)
    onesN = jnp.ones((1, bits.shape[0]), jnp.float32)
    for b in range(30, -1, -1):
        u = res | ((1 << b) - 1)
        cmp = (bits <= u).astype(jnp.float32)
        if axis == 0:
            cnt = _dot(onesN, cmp)
        else:
            cnt = jnp.sum(cmp, axis=axis, keepdims=True)
        res = jnp.where(cnt >= kf, res, res | (1 << b))
    return res


def _d2_rows(xyz, xyzT, r0, bm):
    acc = None
    for c in range(3):
        xi = xyz[0, pl.ds(r0, bm), c:c + 1]
        xj = xyzT[0, c:c + 1, :]
        d = xi - xj
        acc = d * d if acc is None else acc + d * d
    return acc


def _d2_cols(xyz, xyzT, c0, bw):
    acc = None
    for c in range(3):
        xi = xyz[0, :, c:c + 1]
        xj = xyzT[0, c:c + 1, pl.ds(c0, bw)]
        d = xi - xj
        acc = d * d if acc is None else acc + d * d
    return acc


def _row_normalize(x):
    return x / (jnp.sqrt(jnp.sum(x * x, axis=1, keepdims=True)) + NEPS)


def _softmax_rows(x):
    m = jnp.max(x, axis=1, keepdims=True)
    e = jnp.exp(x - m)
    return e / jnp.sum(e, axis=1, keepdims=True)


def _tsearch_pallas(B, N, bm, interpret):
    NB = N // bm

    def body(xyz, xyzT, t_o):
        m = pl.program_id(1)
        r0 = pl.multiple_of(m * bm, bm)
        d2c = _d2_cols(xyz, xyzT, r0, bm)
        bits = jax.lax.bitcast_convert_type(d2c, jnp.int32)
        t_o[0, 0:1, pl.ds(r0, bm)] = _kth_bits(bits, KNN, axis=0)

    return pl.pallas_call(
        body,
        grid=(B, NB),
        in_specs=[pl.BlockSpec((1, N, 3), lambda b, m: (b, 0, 0)),
                  pl.BlockSpec((1, 3, N), lambda b, m: (b, 0, 0))],
        out_specs=pl.BlockSpec((1, 1, N), lambda b, m: (b, 0, 0)),
        out_shape=jax.ShapeDtypeStruct((B, 1, N), jnp.int32),
        interpret=interpret,
    )


def _scale_pallas(B, N, C, H, K, bm, radius2, mode, interpret):
    NB = N // bm
    NPH = 4 if mode == 'scale' else 3
    ones_r = None

    def body(xyz, xyzT, fea, t_row, t_col, W1, b1, W2, b2, *rest):
        if mode == 'scale':
            (pooled_o, cxyz_o, den_o, loss_o,
             fn_s, a2_s, d_s, dinv_s, A_s, acc_s,
             s_s, denom_s, cxyznum_s, pooled_s) = rest
        else:
            (loss_o,
             fn_s, a2_s, d_s, dinv_s, A_s, acc_s) = rest

        p = pl.program_id(1)
        m = pl.program_id(2)
        r0 = pl.multiple_of(m * bm, bm)
        rows = pl.ds(r0, bm)
        ones128 = jnp.ones((1, 128), jnp.float32)

        @pl.when((p == 0) & (m == 0))
        def _init():
            d_s[...] = jnp.zeros((1, N), jnp.float32)
            acc_s[...] = jnp.zeros((8, 128), jnp.float32)
            if mode == 'scale':
                denom_s[...] = jnp.zeros((1, K), jnp.float32)
                cxyznum_s[...] = jnp.zeros((8, K), jnp.float32)
                pooled_s[...] = jnp.zeros((K, C), jnp.float32)

        @pl.when(p == 0)
        def _p0():
            fea_blk = fea[0, rows, :]
            fn_blk = _row_normalize(fea_blk)
            fn_s[rows, :] = fn_blk
            onesC = jnp.ones((1, C), jnp.float32)
            a2_s[0:1, rows] = _dot_nt(onesC, fn_blk * fn_blk, _HI)

        @pl.when(p == 1)
        def _p1():
            d2r = _d2_rows(xyz, xyzT, r0, bm)
            bits = jax.lax.bitcast_convert_type(d2r, jnp.int32)
            t_i = t_col[0, rows, 0:1]
            mi = (bits <= t_i).astype(jnp.float32)
            mj = (bits <= t_row[0, 0:1, :]).astype(jnp.float32)
            inr = (d2r <= radius2).astype(jnp.float32)
            rid = jax.lax.broadcasted_iota(jnp.int32, (bm, N), 0) + r0
            cid = jax.lax.broadcasted_iota(jnp.int32, (bm, N), 1)
            off = (rid != cid).astype(jnp.float32)
            fn_blk = fn_s[rows, :]
            a2i = jnp.sum(fn_blk * fn_blk, axis=1, keepdims=True)
            ab = _dot_nt(fn_blk, fn_s[...])
            fd2 = jnp.maximum((a2i - 2.0 * ab) + a2_s[0:1, :], 0.0)
            A_blk = (jnp.exp(-GAMMA * fd2) * (0.5 * (mi + mj)) * inr * off)
            A_s[rows, :] = A_blk
            d_s[0:1, :] += jnp.sum(A_blk, axis=0, keepdims=True)
            if mode == 'scale':
                fea_blk = fea[0, rows, :]
                h = jnp.maximum(_dot(fea_blk, W1[...]) + b1[0:1, :], 0.0)
                logits = (_dot(h, W2[...]) + b2[0:1, :]) / TAU
                sb = _softmax_rows(logits)
                s_s[rows, :] = sb
                denom_s[0:1, :] += jnp.sum(sb, axis=0, keepdims=True)
                xyzT_blk = xyzT[0, :, rows]
                cxyznum_s[0:3, :] += _dot(xyzT_blk, sb, _HI)

        @pl.when(p == 2)
        def _p2():
            @pl.when(m == 0)
            def _():
                dinv_s[...] = 1.0 / jnp.sqrt(d_s[...] + EPS2)
            A_blk = A_s[rows, :]
            di = 1.0 / jnp.sqrt(jnp.sum(A_blk, axis=1, keepdims=True)
                                + EPS2)
            na = A_blk * di * dinv_s[0:1, :]
            if mode == 'scale':
                xp = _dot(na, fea[0])
                sb = s_s[rows, :]
                pooled_s[...] += _dot_t(sb, xp)
            else:
                d2r = _d2_rows(xyz, xyzT, r0, bm)
                val = jnp.sum(na * d2r)
                acc_s[0:1, :] += val * ones128

                @pl.when(m == NB - 1)
                def _():
                    loss_o[0] = acc_s[...]

        if mode == 'scale':
            @pl.when(p == 3)
            def _p3():
                den = denom_s[0:1, :] + EPS2
                c3 = cxyznum_s[0:3, :] / den
                acc = None
                for c in range(3):
                    pi = xyz[0, rows, c:c + 1]
                    cj = c3[c:c + 1, :]
                    d = pi - cj
                    acc = d * d if acc is None else acc + d * d
                d2pc = acc
                sb = s_s[rows, :]
                a_cs = jnp.sum(sb * d2pc)
                wmin = jnp.min(d2pc, axis=1, keepdims=True)
                w_cs = jnp.sum(1.0 - jnp.exp(-wmin / 2.0))
                sd = sb / den
                c2p = _dot(sd, pooled_s[...])
                c2pn = _row_normalize(c2p)
                fnb = fn_s[rows, :]
                df = c2pn - fnb
                ss_cs = jnp.sum(df * df)
                acc_s[0:1, :] += a_cs * ones128
                acc_s[1:2, :] += w_cs * ones128
                acc_s[2:3, :] += ss_cs * ones128

                @pl.when(m == NB - 1)
                def _():
                    pooled_o[0] = pooled_s[...]
                    cxyz_o[0] = cxyznum_s[...]
                    den_o[0] = denom_s[...]
                    loss_o[0] = acc_s[...]

    f32 = jnp.float32
    full = lambda shp: pl.BlockSpec(shp, lambda b, p, m: (b, 0, 0))
    w2d = lambda shp: pl.BlockSpec(shp, lambda b, p, m: (0, 0))
    in_specs = [
        full((1, N, 3)), full((1, 3, N)), full((1, N, C)),
        full((1, 1, N)), full((1, N, 1)),
        w2d((C, H)), w2d((1, H)), w2d((H, K)), w2d((1, K)),
    ]
    scratch = [
        pltpu.VMEM((N, C), f32),
        pltpu.VMEM((1, N), f32),
        pltpu.VMEM((1, N), f32),
        pltpu.VMEM((1, N), f32),
        pltpu.VMEM((N, N), f32),
        pltpu.VMEM((8, 128), f32),
    ]
    if mode == 'scale':
        scratch += [
            pltpu.VMEM((N, K), f32),
            pltpu.VMEM((1, K), f32),
            pltpu.VMEM((8, K), f32),
            pltpu.VMEM((K, C), f32),
        ]
        out_shape = [
            jax.ShapeDtypeStruct((B, K, C), f32),
            jax.ShapeDtypeStruct((B, 8, K), f32),
            jax.ShapeDtypeStruct((B, 1, K), f32),
            jax.ShapeDtypeStruct((B, 8, 128), f32),
        ]
        out_specs = [full((1, K, C)), full((1, 8, K)),
                     full((1, 1, K)), full((1, 8, 128))]
    else:
        out_shape = [jax.ShapeDtypeStruct((B, 8, 128), f32)]
        out_specs = [full((1, 8, 128))]

    return pl.pallas_call(
        body,
        grid=(B, NPH, NB),
        in_specs=in_specs,
        out_specs=out_specs,
        out_shape=out_shape,
        scratch_shapes=scratch,
        interpret=interpret,
    )


def _global_pallas(B, C, H, Kg, interpret):
    N = 128

    def body(cfea, cxyzP, cxyzT, gW1, gb1, gW2, gb2,
             pooled_o, den_o, loss_o):
        x = cfea[0]
        fnn = _row_normalize(x)
        sim = _dot_nt(fnn, fnn)
        acc = None
        for c in range(3):
            xi = cxyzP[0, :, c:c + 1]
            xj = cxyzT[0, c:c + 1, :]
            d = xi - xj
            acc = d * d if acc is None else acc + d * d
        bits = jax.lax.bitcast_convert_type(acc, jnp.int32)
        t_i = _kth_bits(bits, 8, axis=1)
        t_j = _kth_bits(bits, 8, axis=0)
        mi = (bits <= t_i).astype(jnp.float32)
        mj = (bits <= t_j).astype(jnp.float32)
        rid = jax.lax.broadcasted_iota(jnp.int32, (N, N), 0)
        cid = jax.lax.broadcasted_iota(jnp.int32, (N, N), 1)
        off = (rid != cid).astype(jnp.float32)
        A = jnp.maximum(sim, 0.0) * (0.5 * (mi + mj)) * off
        di = 1.0 / jnp.sqrt(jnp.sum(A, axis=1, keepdims=True) + EPS2)
        dj = 1.0 / jnp.sqrt(jnp.sum(A, axis=0, keepdims=True) + EPS2)
        na = A * di * dj
        h = jnp.maximum(_dot(x, gW1[...]) + gb1[0:1, :], 0.0)
        logits = (_dot(h, gW2[...]) + gb2[0:1, :]) / TAU
        sg = _softmax_rows(logits)
        xp = _dot(na, x)
        pooled_o[0] = _dot_t(sg, xp)
        deng = jnp.sum(sg, axis=0, keepdims=True)
        den_o[0] = deng
        cgnum = _dot(cxyzT[0, 0:3, :], sg, _HI)
        cg = cgnum / (deng + EPS2)
        acc2 = None
        for c in range(3):
            pi = cxyzP[0, :, c:c + 1]
            cj = cg[c:c + 1, :]
            d = pi - cj
            acc2 = d * d if acc2 is None else acc2 + d * d
        asum = jnp.sum(sg * acc2)
        loss_o[0] = asum * jnp.ones((8, 128), jnp.float32)

    f32 = jnp.float32
    full = lambda shp: pl.BlockSpec(shp, lambda b: (b, 0, 0))
    w2d = lambda shp: pl.BlockSpec(shp, lambda b: (0, 0))
    return pl.pallas_call(
        body,
        grid=(B,),
        in_specs=[full((1, N, C)), full((1, N, 3)), full((1, 8, N)),
                  w2d((C, H)), w2d((1, H)), w2d((H, Kg)), w2d((1, Kg))],
        out_specs=[full((1, Kg, C)), full((1, 1, Kg)), full((1, 8, 128))],
        out_shape=[jax.ShapeDtypeStruct((B, Kg, C), f32),
                   jax.ShapeDtypeStruct((B, 1, Kg), f32),
                   jax.ShapeDtypeStruct((B, 8, 128), f32)],
        interpret=interpret,
    )


def _attn_pallas(B, N, C, Kg, n_heads, interpret):
    dh = C // n_heads

    def body(gfea, cfg, Wq, Wk, Wv, Wo, agg_o, loss_o, out_s):
        x = gfea[0]
        q = _dot(x, Wq[...])
        cf = cfg[0]
        kk = _dot(cf, Wk[...])
        vv = _dot(cf, Wv[...])
        for hh in range(n_heads):
            sl = slice(hh * dh, (hh + 1) * dh)
            sc = _dot_nt(q[:, sl], kk[:, sl]) / (float(dh) ** 0.5)
            at = _softmax_rows(sc)
            out_s[:, sl] = _dot(at, vv[:, sl])
        agg_v = _dot(out_s[...], Wo[...])
        agg_o[0] = agg_v
        an = _row_normalize(agg_v)
        gn = _row_normalize(x)
        df = an - gn
        rn = jnp.sqrt(jnp.sum(df * df, axis=1, keepdims=True))
        loss_o[0] = jnp.sum(rn) * jnp.ones((8, 128), jnp.float32)

    f32 = jnp.float32
    full = lambda shp: pl.BlockSpec(shp, lambda b: (b, 0, 0))
    w2d = lambda shp: pl.BlockSpec(shp, lambda b: (0, 0))
    return pl.pallas_call(
        body,
        grid=(B,),
        in_specs=[full((1, N, C)), full((1, Kg, C)),
                  w2d((C, C)), w2d((C, C)), w2d((C, C)), w2d((C, C))],
        out_specs=[full((1, N, C)), full((1, 8, 128))],
        out_shape=[jax.ShapeDtypeStruct((B, N, C), f32),
                   jax.ShapeDtypeStruct((B, 8, 128), f32)],
        scratch_shapes=[pltpu.VMEM((N, C), f32)],
        interpret=interpret,
    )


def _pipeline(g_fea, g_xyz, p_fea_0, p_xyz_0, p_fea_1, p_xyz_1,
              lW1, lb1, lW2, lb2, gW1, gb1, gW2, gb2,
              Wq, Wk, Wv, Wo, interpret=False):
    B, Ng, C = g_fea.shape
    H = lW1.shape[1]
    K = lW2.shape[1]
    Kg = gW2.shape[1]
    bm = 256
    f32 = jnp.float32

    lb1r = lb1.reshape(1, H)
    lb2r = lb2.reshape(1, K)
    gb1r = gb1.reshape(1, H)
    gb2r = gb2.reshape(1, Kg)

    scales = []
    l_loss = jnp.float32(0.0)
    for p_xyz, p_fea in ((p_xyz_0, p_fea_0), (p_xyz_1, p_fea_1)):
        N = p_xyz.shape[1]
        xyzT = jnp.swapaxes(p_xyz, 1, 2)
        t_row = _tsearch_pallas(B, N, bm, interpret)(p_xyz, xyzT)
        t_col = jnp.swapaxes(t_row, 1, 2)
        call = _scale_pallas(B, N, C, H, K, bm, 1.0, 'scale', interpret)
        pooled, cxyznum, denom, lpart = call(
            p_xyz, xyzT, p_fea, t_row, t_col, lW1, lb1r, lW2, lb2r)
        denT = jnp.swapaxes(denom, 1, 2) + EPS2
        c_fea = pooled / denT
        cxyz_cols = cxyznum / (denom + EPS2)
        c_xyz = jnp.swapaxes(cxyz_cols[:, :3, :], 1, 2)
        scales.append((c_fea, c_xyz, cxyz_cols))
        a_sum = jnp.sum(lpart[:, 0, 0])
        w_sum = jnp.sum(lpart[:, 1, 0])
        ss_sum = jnp.sum(lpart[:, 2, 0])
        cut_l = a_sum / (B * N)
        wel = jnp.sqrt(ss_sum) + w_sum / (B * N)
        l_loss = l_loss + cut_l + wel

    c_fea = jnp.concatenate([scales[0][0], scales[1][0]], axis=1)
    c_xyz = jnp.concatenate([scales[0][1], scales[1][1]], axis=1)
    cxyzT = jnp.concatenate([scales[0][2], scales[1][2]], axis=2)

    gcall = _global_pallas(B, C, H, Kg, interpret)
    pooledg, deng, gpart = gcall(c_fea, c_xyz, cxyzT,
                                 gW1, gb1r, gW2, gb2r)
    c_fea_g = pooledg / (jnp.swapaxes(deng, 1, 2) + EPS2)
    cut_loss_c = jnp.sum(gpart[:, 0, 0]) / (B * c_fea.shape[1])

    acall = _attn_pallas(B, Ng, C, Kg, 4, interpret)
    agg, apart = acall(g_fea, c_fea_g, Wq, Wk, Wv, Wo)

    gxyzT = jnp.swapaxes(g_xyz, 1, 2)
    gt_row = _tsearch_pallas(B, Ng, bm, interpret)(g_xyz, gxyzT)
    gt_col = jnp.swapaxes(gt_row, 1, 2)
    ccall = _scale_pallas(B, Ng, C, H, K, bm, 16.0, 'cut', interpret)
    (cpart,) = ccall(g_xyz, gxyzT, g_fea, gt_row, gt_col,
                     lW1, lb1r, lW2, lb2r)
    cut_g = 1e-4 * (jnp.sum(cpart[:, 0, 0]) / B)

    g_loss = jnp.sum(apart[:, 0, 0]) / B + cut_g
    return agg, l_loss + cut_loss_c, g_loss


def kernel(g_fea, g_xyz, p_fea_0, p_xyz_0, p_fea_1, p_xyz_1,
           lW1, lb1, lW2, lb2, gW1, gb1, gW2, gb2, Wq, Wk, Wv, Wo):
    return _pipeline(g_fea, g_xyz, p_fea_0, p_xyz_0, p_fea_1, p_xyz_1,
                     lW1, lb1, lW2, lb2, gW1, gb1, gW2, gb2,
                     Wq, Wk, Wv, Wo)


# speedup vs baseline: 1.4463x; 1.4463x over previous
import functools

import jax
import jax.numpy as jnp
from jax.experimental import pallas as pl
from jax.experimental.pallas import tpu as pltpu

EPS2 = 1e-4
NEPS = 1e-12
TAU = 0.1
GAMMA = 64.0
KNN = 16

_HI = jax.lax.Precision.HIGHEST


def _dot(a, b, prec=None):
    return jax.lax.dot_general(a, b, (((1,), (0,)), ((), ())),
                               precision=prec,
                               preferred_element_type=jnp.float32)


def _dot_t(a, b, prec=None):
    return jax.lax.dot_general(a, b, (((0,), (0,)), ((), ())),
                               precision=prec,
                               preferred_element_type=jnp.float32)


def _dot_nt(a, b, prec=None):
    return jax.lax.dot_general(a, b, (((1,), (1,)), ((), ())),
                               precision=prec,
                               preferred_element_type=jnp.float32)


def _kth_bits(bits, k, axis):
    shp = list(bits.shape)
    shp[axis] = 1
    res = jnp.zeros(shp, jnp.int32)
    kf = jnp.float32(k)
    for b in range(30, -1, -1):
        u = res | ((1 << b) - 1)
        cnt = jnp.sum((bits <= u).astype(jnp.float32), axis=axis,
                      keepdims=True)
        res = jnp.where(cnt >= kf, res, res | (1 << b))
    return res


def _d2_rows(xyz, xyzT, r0, bm):
    acc = None
    for c in range(3):
        xi = xyz[0, pl.ds(r0, bm), c:c + 1]
        xj = xyzT[0, c:c + 1, :]
        d = xi - xj
        acc = d * d if acc is None else acc + d * d
    return acc


def _d2_cols(xyz, xyzT, c0, bw):
    acc = None
    for c in range(3):
        xi = xyz[0, :, c:c + 1]
        xj = xyzT[0, c:c + 1, pl.ds(c0, bw)]
        d = xi - xj
        acc = d * d if acc is None else acc + d * d
    return acc


def _row_normalize(x):
    return x / (jnp.sqrt(jnp.sum(x * x, axis=1, keepdims=True)) + NEPS)


def _softmax_rows(x):
    m = jnp.max(x, axis=1, keepdims=True)
    e = jnp.exp(x - m)
    return e / jnp.sum(e, axis=1, keepdims=True)


def _tsearch_pallas(B, N, bm, interpret):
    NB = N // bm

    def body(xyz, xyzT, t_o):
        m = pl.program_id(1)
        r0 = pl.multiple_of(m * bm, bm)
        d2c = _d2_cols(xyz, xyzT, r0, bm)
        bits = jax.lax.bitcast_convert_type(d2c, jnp.int32)
        t_o[0, 0:1, pl.ds(r0, bm)] = _kth_bits(bits, KNN, axis=0)

    return pl.pallas_call(
        body,
        grid=(B, NB),
        in_specs=[pl.BlockSpec((1, N, 3), lambda b, m: (b, 0, 0)),
                  pl.BlockSpec((1, 3, N), lambda b, m: (b, 0, 0))],
        out_specs=pl.BlockSpec((1, 1, N), lambda b, m: (b, 0, 0)),
        out_shape=jax.ShapeDtypeStruct((B, 1, N), jnp.int32),
        interpret=interpret,
    )


def _scale_pallas(B, N, C, H, K, bm, radius2, mode, interpret):
    NB = N // bm
    NPH = 4 if mode == 'scale' else 3
    ones_r = None

    def body(xyz, xyzT, fea, t_row, t_col, W1, b1, W2, b2, *rest):
        if mode == 'scale':
            (pooled_o, cxyz_o, den_o, loss_o,
             fn_s, a2_s, d_s, dinv_s, A_s, acc_s,
             s_s, denom_s, cxyznum_s, pooled_s) = rest
        else:
            (loss_o,
             fn_s, a2_s, d_s, dinv_s, A_s, acc_s) = rest

        p = pl.program_id(1)
        m = pl.program_id(2)
        r0 = pl.multiple_of(m * bm, bm)
        rows = pl.ds(r0, bm)
        ones128 = jnp.ones((1, 128), jnp.float32)

        @pl.when((p == 0) & (m == 0))
        def _init():
            d_s[...] = jnp.zeros((1, N), jnp.float32)
            acc_s[...] = jnp.zeros((8, 128), jnp.float32)
            if mode == 'scale':
                denom_s[...] = jnp.zeros((1, K), jnp.float32)
                cxyznum_s[...] = jnp.zeros((8, K), jnp.float32)
                pooled_s[...] = jnp.zeros((K, C), jnp.float32)

        @pl.when(p == 0)
        def _p0():
            fea_blk = fea[0, rows, :]
            fn_blk = _row_normalize(fea_blk)
            fn_s[rows, :] = fn_blk
            onesC = jnp.ones((1, C), jnp.float32)
            a2_s[0:1, rows] = _dot_nt(onesC, fn_blk * fn_blk, _HI)

        @pl.when(p == 1)
        def _p1():
            d2r = _d2_rows(xyz, xyzT, r0, bm)
            bits = jax.lax.bitcast_convert_type(d2r, jnp.int32)
            t_i = t_col[0, rows, 0:1]
            mi = (bits <= t_i).astype(jnp.float32)
            mj = (bits <= t_row[0, 0:1, :]).astype(jnp.float32)
            inr = (d2r <= radius2).astype(jnp.float32)
            rid = jax.lax.broadcasted_iota(jnp.int32, (bm, N), 0) + r0
            cid = jax.lax.broadcasted_iota(jnp.int32, (bm, N), 1)
            off = (rid != cid).astype(jnp.float32)
            fn_blk = fn_s[rows, :]
            a2i = jnp.sum(fn_blk * fn_blk, axis=1, keepdims=True)
            ab = _dot_nt(fn_blk, fn_s[...])
            fd2 = jnp.maximum((a2i - 2.0 * ab) + a2_s[0:1, :], 0.0)
            A_blk = (jnp.exp(-GAMMA * fd2) * (0.5 * (mi + mj)) * inr * off)
            A_s[rows, :] = A_blk
            d_s[0:1, :] += jnp.sum(A_blk, axis=0, keepdims=True)
            if mode == 'scale':
                fea_blk = fea[0, rows, :]
                h = jnp.maximum(_dot(fea_blk, W1[...]) + b1[0:1, :], 0.0)
                logits = (_dot(h, W2[...]) + b2[0:1, :]) / TAU
                sb = _softmax_rows(logits)
                s_s[rows, :] = sb
                denom_s[0:1, :] += jnp.sum(sb, axis=0, keepdims=True)
                xyzT_blk = xyzT[0, :, rows]
                cxyznum_s[0:3, :] += _dot(xyzT_blk, sb, _HI)

        @pl.when(p == 2)
        def _p2():
            @pl.when(m == 0)
            def _():
                dinv_s[...] = 1.0 / jnp.sqrt(d_s[...] + EPS2)
            A_blk = A_s[rows, :]
            di = 1.0 / jnp.sqrt(jnp.sum(A_blk, axis=1, keepdims=True)
                                + EPS2)
            na = A_blk * di * dinv_s[0:1, :]
            if mode == 'scale':
                xp = _dot(na, fea[0])
                sb = s_s[rows, :]
                pooled_s[...] += _dot_t(sb, xp)
            else:
                d2r = _d2_rows(xyz, xyzT, r0, bm)
                val = jnp.sum(na * d2r)
                acc_s[0:1, :] += val * ones128

                @pl.when(m == NB - 1)
                def _():
                    loss_o[0] = acc_s[...]

        if mode == 'scale':
            @pl.when(p == 3)
            def _p3():
                den = denom_s[0:1, :] + EPS2
                c3 = cxyznum_s[0:3, :] / den
                acc = None
                for c in range(3):
                    pi = xyz[0, rows, c:c + 1]
                    cj = c3[c:c + 1, :]
                    d = pi - cj
                    acc = d * d if acc is None else acc + d * d
                d2pc = acc
                sb = s_s[rows, :]
                a_cs = jnp.sum(sb * d2pc)
                wmin = jnp.min(d2pc, axis=1, keepdims=True)
                w_cs = jnp.sum(1.0 - jnp.exp(-wmin / 2.0))
                sd = sb / den
                c2p = _dot(sd, pooled_s[...])
                c2pn = _row_normalize(c2p)
                fnb = fn_s[rows, :]
                df = c2pn - fnb
                ss_cs = jnp.sum(df * df)
                acc_s[0:1, :] += a_cs * ones128
                acc_s[1:2, :] += w_cs * ones128
                acc_s[2:3, :] += ss_cs * ones128

                @pl.when(m == NB - 1)
                def _():
                    pooled_o[0] = pooled_s[...]
                    cxyz_o[0] = cxyznum_s[...]
                    den_o[0] = denom_s[...]
                    loss_o[0] = acc_s[...]

    f32 = jnp.float32
    full = lambda shp: pl.BlockSpec(shp, lambda b, p, m: (b, 0, 0))
    w2d = lambda shp: pl.BlockSpec(shp, lambda b, p, m: (0, 0))
    in_specs = [
        full((1, N, 3)), full((1, 3, N)), full((1, N, C)),
        full((1, 1, N)), full((1, N, 1)),
        w2d((C, H)), w2d((1, H)), w2d((H, K)), w2d((1, K)),
    ]
    scratch = [
        pltpu.VMEM((N, C), f32),
        pltpu.VMEM((1, N), f32),
        pltpu.VMEM((1, N), f32),
        pltpu.VMEM((1, N), f32),
        pltpu.VMEM((N, N), f32),
        pltpu.VMEM((8, 128), f32),
    ]
    if mode == 'scale':
        scratch += [
            pltpu.VMEM((N, K), f32),
            pltpu.VMEM((1, K), f32),
            pltpu.VMEM((8, K), f32),
            pltpu.VMEM((K, C), f32),
        ]
        out_shape = [
            jax.ShapeDtypeStruct((B, K, C), f32),
            jax.ShapeDtypeStruct((B, 8, K), f32),
            jax.ShapeDtypeStruct((B, 1, K), f32),
            jax.ShapeDtypeStruct((B, 8, 128), f32),
        ]
        out_specs = [full((1, K, C)), full((1, 8, K)),
                     full((1, 1, K)), full((1, 8, 128))]
    else:
        out_shape = [jax.ShapeDtypeStruct((B, 8, 128), f32)]
        out_specs = [full((1, 8, 128))]

    return pl.pallas_call(
        body,
        grid=(B, NPH, NB),
        in_specs=in_specs,
        out_specs=out_specs,
        out_shape=out_shape,
        scratch_shapes=scratch,
        interpret=interpret,
    )


def _global_pallas(B, C, H, Kg, interpret):
    N = 128

    def body(cfea, cxyzP, cxyzT, gW1, gb1, gW2, gb2,
             pooled_o, den_o, loss_o):
        x = cfea[0]
        fnn = _row_normalize(x)
        sim = _dot_nt(fnn, fnn)
        acc = None
        for c in range(3):
            xi = cxyzP[0, :, c:c + 1]
            xj = cxyzT[0, c:c + 1, :]
            d = xi - xj
            acc = d * d if acc is None else acc + d * d
        bits = jax.lax.bitcast_convert_type(acc, jnp.int32)
        t_i = _kth_bits(bits, 8, axis=1)
        t_j = _kth_bits(bits, 8, axis=0)
        mi = (bits <= t_i).astype(jnp.float32)
        mj = (bits <= t_j).astype(jnp.float32)
        rid = jax.lax.broadcasted_iota(jnp.int32, (N, N), 0)
        cid = jax.lax.broadcasted_iota(jnp.int32, (N, N), 1)
        off = (rid != cid).astype(jnp.float32)
        A = jnp.maximum(sim, 0.0) * (0.5 * (mi + mj)) * off
        di = 1.0 / jnp.sqrt(jnp.sum(A, axis=1, keepdims=True) + EPS2)
        dj = 1.0 / jnp.sqrt(jnp.sum(A, axis=0, keepdims=True) + EPS2)
        na = A * di * dj
        h = jnp.maximum(_dot(x, gW1[...]) + gb1[0:1, :], 0.0)
        logits = (_dot(h, gW2[...]) + gb2[0:1, :]) / TAU
        sg = _softmax_rows(logits)
        xp = _dot(na, x)
        pooled_o[0] = _dot_t(sg, xp)
        deng = jnp.sum(sg, axis=0, keepdims=True)
        den_o[0] = deng
        cgnum = _dot(cxyzT[0, 0:3, :], sg, _HI)
        cg = cgnum / (deng + EPS2)
        acc2 = None
        for c in range(3):
            pi = cxyzP[0, :, c:c + 1]
            cj = cg[c:c + 1, :]
            d = pi - cj
            acc2 = d * d if acc2 is None else acc2 + d * d
        asum = jnp.sum(sg * acc2)
        loss_o[0] = asum * jnp.ones((8, 128), jnp.float32)

    f32 = jnp.float32
    full = lambda shp: pl.BlockSpec(shp, lambda b: (b, 0, 0))
    w2d = lambda shp: pl.BlockSpec(shp, lambda b: (0, 0))
    return pl.pallas_call(
        body,
        grid=(B,),
        in_specs=[full((1, N, C)), full((1, N, 3)), full((1, 8, N)),
                  w2d((C, H)), w2d((1, H)), w2d((H, Kg)), w2d((1, Kg))],
        out_specs=[full((1, Kg, C)), full((1, 1, Kg)), full((1, 8, 128))],
        out_shape=[jax.ShapeDtypeStruct((B, Kg, C), f32),
                   jax.ShapeDtypeStruct((B, 1, Kg), f32),
                   jax.ShapeDtypeStruct((B, 8, 128), f32)],
        interpret=interpret,
    )


def _attn_pallas(B, N, C, Kg, n_heads, interpret):
    dh = C // n_heads

    def body(gfea, cfg, Wq, Wk, Wv, Wo, agg_o, loss_o, out_s):
        x = gfea[0]
        q = _dot(x, Wq[...])
        cf = cfg[0]
        kk = _dot(cf, Wk[...])
        vv = _dot(cf, Wv[...])
        for hh in range(n_heads):
            sl = slice(hh * dh, (hh + 1) * dh)
            sc = _dot_nt(q[:, sl], kk[:, sl]) / (float(dh) ** 0.5)
            at = _softmax_rows(sc)
            out_s[:, sl] = _dot(at, vv[:, sl])
        agg_v = _dot(out_s[...], Wo[...])
        agg_o[0] = agg_v
        an = _row_normalize(agg_v)
        gn = _row_normalize(x)
        df = an - gn
        rn = jnp.sqrt(jnp.sum(df * df, axis=1, keepdims=True))
        loss_o[0] = jnp.sum(rn) * jnp.ones((8, 128), jnp.float32)

    f32 = jnp.float32
    full = lambda shp: pl.BlockSpec(shp, lambda b: (b, 0, 0))
    w2d = lambda shp: pl.BlockSpec(shp, lambda b: (0, 0))
    return pl.pallas_call(
        body,
        grid=(B,),
        in_specs=[full((1, N, C)), full((1, Kg, C)),
                  w2d((C, C)), w2d((C, C)), w2d((C, C)), w2d((C, C))],
        out_specs=[full((1, N, C)), full((1, 8, 128))],
        out_shape=[jax.ShapeDtypeStruct((B, N, C), f32),
                   jax.ShapeDtypeStruct((B, 8, 128), f32)],
        scratch_shapes=[pltpu.VMEM((N, C), f32)],
        interpret=interpret,
    )


def _pipeline(g_fea, g_xyz, p_fea_0, p_xyz_0, p_fea_1, p_xyz_1,
              lW1, lb1, lW2, lb2, gW1, gb1, gW2, gb2,
              Wq, Wk, Wv, Wo, interpret=False):
    B, Ng, C = g_fea.shape
    H = lW1.shape[1]
    K = lW2.shape[1]
    Kg = gW2.shape[1]
    bm = 256
    f32 = jnp.float32

    lb1r = lb1.reshape(1, H)
    lb2r = lb2.reshape(1, K)
    gb1r = gb1.reshape(1, H)
    gb2r = gb2.reshape(1, Kg)

    xyzT_0 = jnp.swapaxes(p_xyz_0, 1, 2)
    xyzT_1 = jnp.swapaxes(p_xyz_1, 1, 2)
    gxyzT = jnp.swapaxes(g_xyz, 1, 2)

    t_row_0 = _tsearch_pallas(B, p_xyz_0.shape[1], 512, interpret)(
        p_xyz_0, xyzT_0)
    st_xyz = jnp.concatenate([p_xyz_1, g_xyz], axis=0)
    st_xyzT = jnp.concatenate([xyzT_1, gxyzT], axis=0)
    t_row_s = _tsearch_pallas(2 * B, Ng, 512, interpret)(st_xyz, st_xyzT)
    t_rows = {0: t_row_0, 1: t_row_s[:B], 2: t_row_s[B:]}

    scales = []
    l_loss = jnp.float32(0.0)
    for si, (p_xyz, p_fea, xyzT) in enumerate(
            ((p_xyz_0, p_fea_0, xyzT_0), (p_xyz_1, p_fea_1, xyzT_1))):
        N = p_xyz.shape[1]
        t_row = t_rows[si]
        t_col = jnp.swapaxes(t_row, 1, 2)
        call = _scale_pallas(B, N, C, H, K, bm, 1.0, 'scale', interpret)
        pooled, cxyznum, denom, lpart = call(
            p_xyz, xyzT, p_fea, t_row, t_col, lW1, lb1r, lW2, lb2r)
        denT = jnp.swapaxes(denom, 1, 2) + EPS2
        c_fea = pooled / denT
        cxyz_cols = cxyznum / (denom + EPS2)
        c_xyz = jnp.swapaxes(cxyz_cols[:, :3, :], 1, 2)
        scales.append((c_fea, c_xyz, cxyz_cols))
        a_sum = jnp.sum(lpart[:, 0, 0])
        w_sum = jnp.sum(lpart[:, 1, 0])
        ss_sum = jnp.sum(lpart[:, 2, 0])
        cut_l = a_sum / (B * N)
        wel = jnp.sqrt(ss_sum) + w_sum / (B * N)
        l_loss = l_loss + cut_l + wel

    c_fea = jnp.concatenate([scales[0][0], scales[1][0]], axis=1)
    c_xyz = jnp.concatenate([scales[0][1], scales[1][1]], axis=1)
    cxyzT = jnp.concatenate([scales[0][2], scales[1][2]], axis=2)

    gcall = _global_pallas(B, C, H, Kg, interpret)
    pooledg, deng, gpart = gcall(c_fea, c_xyz, cxyzT,
                                 gW1, gb1r, gW2, gb2r)
    c_fea_g = pooledg / (jnp.swapaxes(deng, 1, 2) + EPS2)
    cut_loss_c = jnp.sum(gpart[:, 0, 0]) / (B * c_fea.shape[1])

    acall = _attn_pallas(B, Ng, C, Kg, 4, interpret)
    agg, apart = acall(g_fea, c_fea_g, Wq, Wk, Wv, Wo)

    gt_row = t_rows[2]
    gt_col = jnp.swapaxes(gt_row, 1, 2)
    ccall = _scale_pallas(B, Ng, C, H, K, bm, 16.0, 'cut', interpret)
    (cpart,) = ccall(g_xyz, gxyzT, g_fea, gt_row, gt_col,
                     lW1, lb1r, lW2, lb2r)
    cut_g = 1e-4 * (jnp.sum(cpart[:, 0, 0]) / B)

    g_loss = jnp.sum(apart[:, 0, 0]) / B + cut_g
    return agg, l_loss + cut_loss_c, g_loss


def kernel(g_fea, g_xyz, p_fea_0, p_xyz_0, p_fea_1, p_xyz_1,
           lW1, lb1, lW2, lb2, gW1, gb1, gW2, gb2, Wq, Wk, Wv, Wo):
    return _pipeline(g_fea, g_xyz, p_fea_0, p_xyz_0, p_fea_1, p_xyz_1,
                     lW1, lb1, lW2, lb2, gW1, gb1, gW2, gb2,
                     Wq, Wk, Wv, Wo)
